# baseline (speedup 1.0000x reference)
import functools
import math

import jax
import jax.numpy as jnp
from jax import lax
from jax.experimental import pallas as pl
from jax.experimental.pallas import tpu as pltpu
from jax.experimental.pallas import tpu_sc as plsc

PI_F = float(math.pi)

_NC = 2
_NS = 16
_L = 16
_NW = _NC * _NS

_PI_2_HI = 1.5707855224609375
_PI_2_LO = 1.0804334124034763e-05
_TWO_OVER_PI = 2.0 / PI_F


def _row_to_channel(r):
    return 2 * r if r < 8 else 2 * (r - 8) + 1


def _floorf(v):
    ti = v.astype(jnp.int32)
    tf = ti.astype(jnp.float32)
    ti = jnp.where(tf > v, ti - 1, ti)
    return ti


def _sw_sincos(x):
    v = x * _TWO_OVER_PI
    v = jnp.clip(v, -1.0e9, 1.0e9)
    ki = _floorf(v + 0.5)
    kf = ki.astype(jnp.float32)
    y = (x - kf * _PI_2_HI) - kf * _PI_2_LO
    y = jnp.clip(y, -0.7855, 0.7855)
    q = jnp.bitwise_and(ki, 3)
    y2 = y * y
    s = y * (1.0 + y2 * (-0.16666667 + y2 * (8.3333310e-3
             + y2 * (-1.9840874e-4 + y2 * 2.7525562e-6))))
    c = 1.0 + y2 * (-0.5 + y2 * (4.1666638e-2
            + y2 * (-1.3888378e-3 + y2 * 2.4760495e-5)))
    swap = (q == 1) | (q == 3)
    sin_b = jnp.where(swap, c, s)
    cos_b = jnp.where(swap, s, c)
    sout = jnp.where((q == 2) | (q == 3), -sin_b, sin_b)
    cout = jnp.where((q == 1) | (q == 2), -cos_b, cos_b)
    return sout, cout


def _sw_sqrt(x):
    i = lax.bitcast_convert_type(x, jnp.int32)
    i = 0x5F3759DF - lax.shift_right_logical(i, 1)
    r = lax.bitcast_convert_type(i, jnp.float32)
    r = r * (1.5 - 0.5 * x * r * r)
    r = r * (1.5 - 0.5 * x * r * r)
    r = r * (1.5 - 0.5 * x * r * r)
    out = x * r
    out = jnp.where(x <= 0.0, 0.0, out)
    return jnp.where(x == jnp.inf, x, out)


def _sw_atan2(y, x):
    ax = jnp.abs(x)
    ay = jnp.abs(y)
    mx = jnp.maximum(ax, ay)
    mn = jnp.minimum(ax, ay)
    r = mn / mx
    r = jnp.where(mx == 0.0, 0.0, r)
    hi = r > 0.4142135623730951
    t = jnp.where(hi, (r - 1.0) / (r + 1.0), r)
    t2 = t * t
    a = t * (1.0 + t2 * (-0.33333072 + t2 * (0.19993057
             + t2 * (-0.14203644 + t2 * 0.10640934))))
    a = jnp.where(hi, (PI_F / 4.0) + a, a)
    a = jnp.where(ay > ax, (PI_F / 2.0) - a, a)
    a = jnp.where(x < 0.0, PI_F - a, a)
    return jnp.where(y < 0.0, -a, a)


@functools.lru_cache(maxsize=None)
def _make_sc_kernel(B, C, H, W, K):
    P = B * K
    assert P % _NW == 0
    n = P // _NW
    assert n % _L == 0
    assert K % n == 0
    assert C == 16
    assert W == 256
    _NBUF = 4
    RAD = PI_F / 180.0
    DEG = 180.0 / PI_F

    mesh = plsc.VectorSubcoreMesh(core_axis_name="c", subcore_axis_name="s")

    @functools.partial(
        pl.kernel,
        mesh=mesh,
        compiler_params=pltpu.CompilerParams(needs_layout_passes=False),
        out_type=jax.ShapeDtypeStruct((_NW * 2 * _L,), jnp.float32),
        scratch_types=[
            pltpu.VMEM((n,), jnp.int32),
            pltpu.VMEM((n,), jnp.float32),
            pltpu.VMEM((16, n), jnp.int32),
            pltpu.VMEM((16, n), jnp.int32),
            pltpu.VMEM((16, n), jnp.float32),
            pltpu.VMEM((16, n), jnp.float32),
            pltpu.VMEM((2 * _L,), jnp.float32),
            pltpu.VMEM((_NBUF, n, W), jnp.float32),
            pltpu.SemaphoreType.DMA,
            pltpu.SemaphoreType.DMA,
            pltpu.SemaphoreType.DMA,
            pltpu.SemaphoreType.DMA,
            pltpu.SemaphoreType.DMA,
        ],
    )
    def sc_kernel(outv, tarf, indf, maskf, part,
                  ind_v, mask_v, idxp, idxt, gp, gt, pbuf, rbuf,
                  semt, s0, s1, s2, s3):
        sems = [s0, s1, s2, s3]
        wid = lax.axis_index("s") * _NC + lax.axis_index("c")
        base = wid * n
        pltpu.sync_copy(indf.at[pl.ds(base, n)], ind_v)
        pltpu.sync_copy(maskf.at[pl.ds(base, n)], mask_v)
        b_s = base // K
        for t in range(n // _L):
            iv = ind_v[pl.ds(t * _L, _L)]
            hv = jnp.right_shift(iv, 8)
            pv = base + t * _L + lax.iota(jnp.int32, _L)
            for r in range(16):
                c = _row_to_channel(r)
                idxp[r, pl.ds(t * _L, _L)] = hv + (b_s * C + c) * H
                idxt[r, pl.ds(t * _L, _L)] = pv * C + c
        tcopies = []
        for r in range(16):
            tcopies.append(
                pltpu.async_copy(tarf.at[idxt.at[r]], gt.at[r], semt))
        copies = [None] * 16
        for r in range(_NBUF):
            copies[r] = pltpu.async_copy(
                outv.at[idxp.at[r]], rbuf.at[r], sems[r])
        for r in range(16):
            copies[r].wait()
            if r + _NBUF < 16:
                copies[r + _NBUF] = pltpu.async_copy(
                    outv.at[idxp.at[r + _NBUF]],
                    rbuf.at[(r + _NBUF) % _NBUF],
                    sems[(r + _NBUF) % _NBUF])
            rb = rbuf.at[r % _NBUF]
            for t in range(n // _L):
                iv = ind_v[pl.ds(t * _L, _L)]
                wv = jnp.bitwise_and(iv, W - 1)
                ivec = t * _L + lax.iota(jnp.int32, _L)
                vals = plsc.load_gather(rb, [ivec, wv])
                gp[r, pl.ds(t * _L, _L)] = vals
        for cp in tcopies:
            cp.wait()

        acc = jnp.zeros((_L,), jnp.float32)
        accm = jnp.zeros((_L,), jnp.float32)
        for t in range(n // _L):
            sl = pl.ds(t * _L, _L)
            mv = mask_v[sl]
            keep = mv != 0.0
            txs, tys = [], []
            for j in range(8):
                dtj = gt[j, sl] * mv
                atj = gt[8 + j, sl] * mv
                s, c = _sw_sincos(atj * RAD)
                txs.append(dtj * c)
                tys.append(dtj * s)
            for j in range(8):
                dpj = gp[j, sl] * mv
                apj = gp[8 + j, sl] * mv
                s, c = _sw_sincos(apj * RAD)
                px = dpj * c
                py = dpj * s
                ax, ay = txs[(j - 1) % 8], tys[(j - 1) % 8]
                bx, by = txs[j], tys[j]
                nx, ny = txs[(j + 1) % 8], tys[(j + 1) % 8]

                def closest_sq(a_x, a_y, b_x, b_y):
                    abx = b_x - a_x
                    aby = b_y - a_y
                    tt = ((px - a_x) * abx + (py - a_y) * aby) \
                        / (abx * abx + aby * aby)
                    tt = jnp.clip(tt, 0.0, 1.0)
                    cx = a_x + tt * abx
                    cy = a_y + tt * aby
                    dx = px - cx
                    dy = py - cy
                    return cx, cy, dx * dx + dy * dy

                c1x, c1y, d1 = closest_sq(ax, ay, bx, by)
                c2x, c2y, d2 = closest_sq(bx, by, nx, ny)
                use2 = d2 < d1
                if j in (0, 3, 7):
                    prx, pry = bx, by
                else:
                    chx = jnp.where(use2, c2x, c1x)
                    chy = jnp.where(use2, c2y, c1y)
                    same_nb = (ax == nx) & (ay == ny)
                    prx = jnp.where(same_nb, bx, chx)
                    pry = jnp.where(same_nb, by, chy)
                prx = jnp.where(keep, prx, 0.0)
                pry = jnp.where(keep, pry, 0.0)
                dist = _sw_sqrt(prx * prx + pry * pry)
                ang = _sw_atan2(pry, prx) * DEG
                ang = jnp.where(ang < 0.0, ang + 360.0, ang)
                acc = acc + jnp.abs(dpj * mv - dist * mv) \
                    + jnp.abs(apj * mv - ang * mv)
            accm = accm + mv
        pbuf[pl.ds(0, _L)] = acc
        pbuf[pl.ds(_L, _L)] = accm
        pltpu.sync_copy(pbuf, part.at[pl.ds(wid * 2 * _L, 2 * _L)])

    return sc_kernel


def kernel(output, mask, ind, target):
    B, C, H, W = output.shape
    K = ind.shape[1]
    P = B * K
    outv = output.reshape(B * C * H, W)
    tarf = target.reshape(P * C)
    indf = ind.reshape(P).astype(jnp.int32)
    maskf = mask.reshape(P)
    part = _make_sc_kernel(B, C, H, W, K)(outv, tarf, indf, maskf)
    part = part.reshape(_NW, 2, _L)
    total = jnp.sum(part[:, 0, :])
    denom = jnp.sum(part[:, 1, :]) * C + 0.0001
    return total / denom
